# baseline (speedup 1.0000x reference)
import jax
import jax.numpy as jnp
from jax import lax
from jax.experimental import pallas as pl
from jax.experimental.pallas import tpu as pltpu
from jax.experimental.pallas import tpu_sc as plsc

NC, NS = 2, 16
NW = NC * NS
B, LS, LL = 4096, 200, 50
NSEG = B * LS
NLINK = B * LL
SEG_PER_W = NSEG // NW
EXT_PER_W = B // NW
CSEG = 640
NSTR = CSEG // 128
SEG_CHUNKS = SEG_PER_W // CSEG


def _take16(vec, idx):
    return vec.at[idx].get(mode="promise_in_bounds")


def _i16(c):
    return jnp.full((16,), c, jnp.int32)


def _body(weekID, timeID, driverID, seg_idx,
          if0, if1, if2, if3, ff0, ff1, ff2, ff3,
          cross_idx, delay,
          Wsmall, Wtime, Wdrv, Wseg, Wcross,
          ext_out, seg_out, link_out,
          t_small,
          widx, tidx, didx, etm, edr, pe,
          sidx, sib, gb, pb,
          lidx, gl, lb,
          sem, isem, gsem, csem, osem):
    wid = lax.axis_index("s") * NC + lax.axis_index("c")
    iota16 = lax.iota(jnp.int32, 16)
    ffs = (ff0, ff1, ff2, ff3)
    ifs = (if0, if1, if2, if3)

    pltpu.sync_copy(Wsmall, t_small)
    tcol = [t_small[k, :] for k in range(11)]

    ebase = wid * EXT_PER_W
    pltpu.sync_copy(weekID.at[pl.ds(ebase, EXT_PER_W)], widx)
    pltpu.sync_copy(timeID.at[pl.ds(ebase, EXT_PER_W)], tidx)
    pltpu.sync_copy(driverID.at[pl.ds(ebase, EXT_PER_W)], didx)
    ecps = [
        pltpu.async_copy(Wtime.at[tidx], etm, sem),
        pltpu.async_copy(Wdrv.at[didx], edr, sem),
    ]
    for c in ecps:
        c.wait()

    def ext_grp(g, c2):
        off = g * 16
        wk = widx[pl.ds(off, 16)]
        for c in range(3):
            pe[c, pl.ds(off, 16)] = _take16(tcol[8 + c], wk)
        for r in range(16):
            col = jnp.full((16,), off + r, jnp.int32)
            plsc.store_scatter(pe, [3 + iota16, col], etm[off + r, :])
            plsc.store_scatter(pe, [8 + iota16, col], edr[off + r, :])
        return c2

    lax.fori_loop(0, EXT_PER_W // 16, ext_grp, 0)

    def ext_out_fire(c, c2):
        pltpu.async_copy(pe.at[c], ext_out.at[c, pl.ds(ebase, EXT_PER_W)],
                         sem)
        return c2

    def ext_out_drain(c, c2):
        pltpu.make_async_copy(pe.at[0], ext_out.at[0, pl.ds(0, EXT_PER_W)],
                              sem).wait()
        return c2

    lax.fori_loop(0, 24, ext_out_fire, 0)
    lax.fori_loop(0, 24, ext_out_drain, 0)

    def fire_inputs(i, o):
        q1 = wid * SEG_PER_W + i * CSEG
        pltpu.async_copy(seg_idx.at[pl.ds(q1, CSEG)], sidx.at[o], isem.at[o])
        for t in range(4):
            pltpu.async_copy(ifs[t].at[pl.ds(q1, CSEG)], sib.at[o, t],
                             isem.at[o])

    def drain_inputs(o):
        pltpu.make_async_copy(seg_idx.at[pl.ds(0, CSEG)], sidx.at[o],
                              isem.at[o]).wait()
        for t in range(4):
            pltpu.make_async_copy(ifs[t].at[pl.ds(0, CSEG)], sib.at[o, t],
                                  isem.at[o]).wait()

    def fire_gathers(o):
        def fg(j, c2):
            pltpu.async_copy(Wseg.at[sidx.at[o, pl.ds(j * 128, 128)]],
                             gb.at[o, pl.ds(j * 128, 128)], gsem.at[o])
            return c2
        lax.fori_loop(0, NSTR, fg, 0)

    def drain_gathers(s):
        def dg(j, c2):
            pltpu.make_async_copy(Wseg.at[pl.ds(0, 128)],
                                  gb.at[s, pl.ds(0, 128)], gsem.at[s]).wait()
            return c2
        lax.fori_loop(0, NSTR, dg, 0)

    q00 = wid * SEG_PER_W
    pltpu.sync_copy(seg_idx.at[pl.ds(q00, CSEG)], sidx.at[0])
    for t in range(4):
        pltpu.sync_copy(ifs[t].at[pl.ds(q00, CSEG)], sib.at[0, t])
    fire_gathers(0)

    def chunk(i, carry):
        s = i % 2
        o = 1 - s
        q0 = wid * SEG_PER_W + i * CSEG

        @pl.when(i >= 2)
        def _():
            def od(c, c2):
                pltpu.make_async_copy(pb.at[0, 0],
                                      seg_out.at[0, pl.ds(0, CSEG)],
                                      osem.at[s]).wait()
                return c2
            lax.fori_loop(0, 28, od, 0)

        for f in range(4):
            pltpu.async_copy(ffs[f].at[pl.ds(q0, CSEG)], pb.at[s, 24 + f],
                             csem.at[s])

        @pl.when(i < SEG_CHUNKS - 1)
        def _():
            fire_inputs(i + 1, o)

        def grp_small(g, c2):
            off = g * 16
            for t in range(4):
                idx = sib[s, t, pl.ds(off, 16)]
                for c in range(2):
                    pb[s, 16 + 2 * t + c, pl.ds(off, 16)] = _take16(
                        tcol[2 * t + c], idx)
            return c2

        lax.fori_loop(0, CSEG // 16, grp_small, 0)
        drain_gathers(s)

        def grp_tr(g, c2):
            off = g * 16
            for r in range(16):
                plsc.store_scatter(
                    pb.at[s], [iota16, jnp.full((16,), off + r, jnp.int32)],
                    gb[s, off + r, :])
            return c2

        lax.fori_loop(0, CSEG // 16, grp_tr, 0)

        def cd(f, c2):
            pltpu.make_async_copy(ffs[0].at[pl.ds(0, CSEG)], pb.at[s, 24],
                                  csem.at[s]).wait()
            return c2

        lax.fori_loop(0, 4, cd, 0)

        def of(c, c2):
            pltpu.async_copy(pb.at[s, c], seg_out.at[c, pl.ds(q0, CSEG)],
                             osem.at[s])
            return c2

        lax.fori_loop(0, 28, of, 0)

        @pl.when(i < SEG_CHUNKS - 1)
        def _():
            drain_inputs(o)
            fire_gathers(o)

        return carry

    lax.fori_loop(0, SEG_CHUNKS, chunk, 0)

    def final_drain(c, c2):
        pltpu.make_async_copy(pb.at[0, 0], seg_out.at[0, pl.ds(0, CSEG)],
                              osem.at[0]).wait()
        pltpu.make_async_copy(pb.at[1, 0], seg_out.at[0, pl.ds(0, CSEG)],
                              osem.at[1]).wait()
        return c2

    lax.fori_loop(0, 28, final_drain, 0)

    bbase = wid * EXT_PER_W

    def link_l(l, carry):
        q0 = l * B + bbase
        pltpu.sync_copy(cross_idx.at[pl.ds(q0, EXT_PER_W)], lidx)
        cp = pltpu.async_copy(Wcross.at[lidx], gl, sem)
        cp.wait()

        def grp(g, c2):
            off = g * 16
            for r in range(16):
                plsc.store_scatter(
                    lb, [iota16, jnp.full((16,), off + r, jnp.int32)],
                    gl[off + r, :])
            return c2

        lax.fori_loop(0, EXT_PER_W // 16, grp, 0)
        dcp = pltpu.async_copy(delay.at[pl.ds(q0, EXT_PER_W)],
                               lb.at[15], sem)
        dcp.wait()

        def lf(c, c2):
            pltpu.async_copy(
                lb.at[c], link_out.at[l * 16 + c, pl.ds(bbase, EXT_PER_W)],
                sem)
            return c2

        def ld(c, c2):
            pltpu.make_async_copy(lb.at[0],
                                  link_out.at[0, pl.ds(0, EXT_PER_W)],
                                  sem).wait()
            return c2

        lax.fori_loop(0, 16, lf, 0)
        lax.fori_loop(0, 16, ld, 0)
        return carry

    lax.fori_loop(0, LL, link_l, 0)


@jax.jit
def _run(weekID, timeID, driverID, seg_idx,
         if0, if1, if2, if3, ff0, ff1, ff2, ff3,
         cross_idx, delay, Wsmall, Wtime, Wdrv, Wseg, Wcross):
    mesh = plsc.VectorSubcoreMesh(core_axis_name="c", subcore_axis_name="s",
                                  num_cores=NC, num_subcores=NS)
    f = pl.kernel(
        _body,
        out_type=(
            jax.ShapeDtypeStruct((24, B), jnp.float32),
            jax.ShapeDtypeStruct((28, NSEG), jnp.float32),
            jax.ShapeDtypeStruct((LL * 16, B), jnp.float32),
        ),
        mesh=mesh,
        compiler_params=pltpu.CompilerParams(needs_layout_passes=False,
                                             use_tc_tiling_on_sc=False),
        scratch_types=[
            pltpu.VMEM((11, 16), jnp.float32),
            pltpu.VMEM((EXT_PER_W,), jnp.int32),
            pltpu.VMEM((EXT_PER_W,), jnp.int32),
            pltpu.VMEM((EXT_PER_W,), jnp.int32),
            pltpu.VMEM((EXT_PER_W, 16), jnp.float32),
            pltpu.VMEM((EXT_PER_W, 16), jnp.float32),
            pltpu.VMEM((24, EXT_PER_W), jnp.float32),
            pltpu.VMEM((2, CSEG), jnp.int32),
            pltpu.VMEM((2, 4, CSEG), jnp.int32),
            pltpu.VMEM((2, CSEG, 16), jnp.float32),
            pltpu.VMEM((2, 28, CSEG), jnp.float32),
            pltpu.VMEM((EXT_PER_W,), jnp.int32),
            pltpu.VMEM((EXT_PER_W, 16), jnp.float32),
            pltpu.VMEM((16, EXT_PER_W), jnp.float32),
            pltpu.SemaphoreType.DMA,
            pltpu.SemaphoreType.DMA((2,)),
            pltpu.SemaphoreType.DMA((2,)),
            pltpu.SemaphoreType.DMA((2,)),
            pltpu.SemaphoreType.DMA((2,)),
        ],
    )
    return f(weekID, timeID, driverID, seg_idx, if0, if1, if2, if3,
             ff0, ff1, ff2, ff3, cross_idx, delay,
             Wsmall, Wtime, Wdrv, Wseg, Wcross)


def kernel(weekID, timeID, driverID, segID, segment_functional_level,
           roadState, laneNum, roadLevel, wid, speedLimit, time, len,
           crossID, delayTime, W_weekID, W_timeID, W_driverID, W_segID,
           W_segment_functional_level, W_roadState, W_laneNum, W_roadLevel,
           W_crossID):
    def qv(x):
        return x.T.reshape(-1)

    def padcols(w):
        n, d = w.shape
        wp = jnp.pad(w, ((0, 16 - n), (0, 0)))
        return [wp[:, c] for c in range(d)]

    Wsmall = jnp.stack(
        padcols(W_segment_functional_level) + padcols(W_roadState)
        + padcols(W_laneNum) + padcols(W_roadLevel)
        + padcols(W_weekID))

    def rowmajor(w):
        return lax.optimization_barrier(w.reshape(-1)).reshape(w.shape)

    ext_t, seg_t, link_t = _run(
        weekID.astype(jnp.int32), timeID.astype(jnp.int32),
        driverID.astype(jnp.int32), qv(segID.astype(jnp.int32)),
        qv(segment_functional_level.astype(jnp.int32)),
        qv(roadState.astype(jnp.int32)), qv(laneNum.astype(jnp.int32)),
        qv(roadLevel.astype(jnp.int32)),
        qv(wid), qv(speedLimit), qv(time), qv(len),
        qv(crossID.astype(jnp.int32)), qv(delayTime),
        Wsmall, jnp.pad(W_timeID, ((0, 0), (0, 11))), rowmajor(W_driverID),
        rowmajor(W_segID), jnp.pad(W_crossID, ((0, 0), (0, 1))))

    ext = ext_t.T
    seg = seg_t.reshape(28, LS, B).transpose(2, 1, 0)
    link = link_t.reshape(LL, 16, B).transpose(2, 0, 1)
    return ext, seg, link


# speedup vs baseline: 1.0058x; 1.0058x over previous
import jax
import jax.numpy as jnp
from jax import lax
from jax.experimental import pallas as pl
from jax.experimental.pallas import tpu as pltpu
from jax.experimental.pallas import tpu_sc as plsc

NC, NS = 2, 16
NW = NC * NS
B, LS, LL = 4096, 200, 50
NSEG = B * LS
NLINK = B * LL
SEG_PER_W = NSEG // NW
EXT_PER_W = B // NW
CSEG = 1024
NSTR = CSEG // 128
SEG_CHUNKS = SEG_PER_W // CSEG


def _take16(vec, idx):
    return vec.at[idx].get(mode="promise_in_bounds")


def _i16(c):
    return jnp.full((16,), c, jnp.int32)


def _body(weekID, timeID, driverID, seg_idx,
          if0, if1, if2, if3, ff0, ff1, ff2, ff3,
          cross_idx, delay,
          Wsmall, Wtime, Wdrv, Wseg, Wcross,
          ext_out, seg_out, link_out,
          t_small,
          widx, tidx, didx, etm, edr, pe,
          sidx, sib, gb, pb,
          lidx, gl, lb,
          sem, isem, gsem, csem, osem):
    wid = lax.axis_index("s") * NC + lax.axis_index("c")
    iota16 = lax.iota(jnp.int32, 16)
    ffs = (ff0, ff1, ff2, ff3)
    ifs = (if0, if1, if2, if3)

    pltpu.sync_copy(Wsmall, t_small)
    tcol = [t_small[k, :] for k in range(11)]

    ebase = wid * EXT_PER_W
    pltpu.sync_copy(weekID.at[pl.ds(ebase, EXT_PER_W)], widx)
    pltpu.sync_copy(timeID.at[pl.ds(ebase, EXT_PER_W)], tidx)
    pltpu.sync_copy(driverID.at[pl.ds(ebase, EXT_PER_W)], didx)
    ecps = [
        pltpu.async_copy(Wtime.at[tidx], etm, sem),
        pltpu.async_copy(Wdrv.at[didx], edr, sem),
    ]
    for c in ecps:
        c.wait()

    def ext_grp(g, c2):
        off = g * 16
        wk = widx[pl.ds(off, 16)]
        for c in range(3):
            pe[c, pl.ds(off, 16)] = _take16(tcol[8 + c], wk)
        for r in range(16):
            col = jnp.full((16,), off + r, jnp.int32)
            plsc.store_scatter(pe, [3 + iota16, col], etm[off + r, :])
            plsc.store_scatter(pe, [8 + iota16, col], edr[off + r, :])
        return c2

    lax.fori_loop(0, EXT_PER_W // 16, ext_grp, 0)

    def ext_out_fire(c, c2):
        pltpu.async_copy(pe.at[c], ext_out.at[c, pl.ds(ebase, EXT_PER_W)],
                         sem)
        return c2

    def ext_out_drain(c, c2):
        pltpu.make_async_copy(pe.at[0], ext_out.at[0, pl.ds(0, EXT_PER_W)],
                              sem).wait()
        return c2

    lax.fori_loop(0, 24, ext_out_fire, 0)
    lax.fori_loop(0, 24, ext_out_drain, 0)

    def fire_inputs(i, o):
        q1 = wid * SEG_PER_W + i * CSEG
        pltpu.async_copy(seg_idx.at[pl.ds(q1, CSEG)], sidx.at[o], isem.at[o])
        for t in range(4):
            pltpu.async_copy(ifs[t].at[pl.ds(q1, CSEG)], sib.at[o, t],
                             isem.at[o])

    def drain_inputs(o):
        pltpu.make_async_copy(seg_idx.at[pl.ds(0, CSEG)], sidx.at[o],
                              isem.at[o]).wait()
        for t in range(4):
            pltpu.make_async_copy(ifs[t].at[pl.ds(0, CSEG)], sib.at[o, t],
                                  isem.at[o]).wait()

    def fire_gathers(o):
        def fg(j, c2):
            pltpu.async_copy(Wseg.at[sidx.at[o, pl.ds(j * 128, 128)]],
                             gb.at[o, pl.ds(j * 128, 128)], gsem.at[o])
            return c2
        lax.fori_loop(0, NSTR, fg, 0)

    def drain_gathers(s):
        def dg(j, c2):
            pltpu.make_async_copy(Wseg.at[pl.ds(0, 128)],
                                  gb.at[s, pl.ds(0, 128)], gsem.at[s]).wait()
            return c2
        lax.fori_loop(0, NSTR, dg, 0)

    q00 = wid * SEG_PER_W
    pltpu.sync_copy(seg_idx.at[pl.ds(q00, CSEG)], sidx.at[0])
    for t in range(4):
        pltpu.sync_copy(ifs[t].at[pl.ds(q00, CSEG)], sib.at[0, t])
    fire_gathers(0)

    def chunk(i, carry):
        s = i % 2
        o = 1 - s
        q0 = wid * SEG_PER_W + i * CSEG

        @pl.when(i >= 2)
        def _():
            def od(c, c2):
                pltpu.make_async_copy(pb.at[0, 0],
                                      seg_out.at[0, pl.ds(0, CSEG)],
                                      osem.at[s]).wait()
                return c2
            lax.fori_loop(0, 28, od, 0)

        for f in range(4):
            pltpu.async_copy(ffs[f].at[pl.ds(q0, CSEG)], pb.at[s, 24 + f],
                             csem.at[s])

        @pl.when(i < SEG_CHUNKS - 1)
        def _():
            fire_inputs(i + 1, o)

        def grp_small(g, c2):
            off = g * 16
            for t in range(4):
                idx = sib[s, t, pl.ds(off, 16)]
                for c in range(2):
                    pb[s, 16 + 2 * t + c, pl.ds(off, 16)] = _take16(
                        tcol[2 * t + c], idx)
            return c2

        lax.fori_loop(0, CSEG // 16, grp_small, 0)
        drain_gathers(s)

        def grp_tr(g, c2):
            off = g * 16
            for r in range(16):
                plsc.store_scatter(
                    pb.at[s], [iota16, jnp.full((16,), off + r, jnp.int32)],
                    gb[s, off + r, :])
            return c2

        lax.fori_loop(0, CSEG // 16, grp_tr, 0)

        def cd(f, c2):
            pltpu.make_async_copy(ffs[0].at[pl.ds(0, CSEG)], pb.at[s, 24],
                                  csem.at[s]).wait()
            return c2

        lax.fori_loop(0, 4, cd, 0)

        def of(c, c2):
            pltpu.async_copy(pb.at[s, c], seg_out.at[c, pl.ds(q0, CSEG)],
                             osem.at[s])
            return c2

        lax.fori_loop(0, 28, of, 0)

        @pl.when(i < SEG_CHUNKS - 1)
        def _():
            drain_inputs(o)
            fire_gathers(o)

        return carry

    lax.fori_loop(0, SEG_CHUNKS, chunk, 0)

    def final_drain(c, c2):
        pltpu.make_async_copy(pb.at[0, 0], seg_out.at[0, pl.ds(0, CSEG)],
                              osem.at[0]).wait()
        pltpu.make_async_copy(pb.at[1, 0], seg_out.at[0, pl.ds(0, CSEG)],
                              osem.at[1]).wait()
        return c2

    lax.fori_loop(0, 28, final_drain, 0)

    bbase = wid * EXT_PER_W

    def link_l(l, carry):
        q0 = l * B + bbase
        pltpu.sync_copy(cross_idx.at[pl.ds(q0, EXT_PER_W)], lidx)
        cp = pltpu.async_copy(Wcross.at[lidx], gl, sem)
        cp.wait()

        def grp(g, c2):
            off = g * 16
            for r in range(16):
                plsc.store_scatter(
                    lb, [iota16, jnp.full((16,), off + r, jnp.int32)],
                    gl[off + r, :])
            return c2

        lax.fori_loop(0, EXT_PER_W // 16, grp, 0)
        dcp = pltpu.async_copy(delay.at[pl.ds(q0, EXT_PER_W)],
                               lb.at[15], sem)
        dcp.wait()

        def lf(c, c2):
            pltpu.async_copy(
                lb.at[c], link_out.at[l * 16 + c, pl.ds(bbase, EXT_PER_W)],
                sem)
            return c2

        def ld(c, c2):
            pltpu.make_async_copy(lb.at[0],
                                  link_out.at[0, pl.ds(0, EXT_PER_W)],
                                  sem).wait()
            return c2

        lax.fori_loop(0, 16, lf, 0)
        lax.fori_loop(0, 16, ld, 0)
        return carry

    lax.fori_loop(0, LL, link_l, 0)


@jax.jit
def _run(weekID, timeID, driverID, seg_idx,
         if0, if1, if2, if3, ff0, ff1, ff2, ff3,
         cross_idx, delay, Wsmall, Wtime, Wdrv, Wseg, Wcross):
    mesh = plsc.VectorSubcoreMesh(core_axis_name="c", subcore_axis_name="s",
                                  num_cores=NC, num_subcores=NS)
    f = pl.kernel(
        _body,
        out_type=(
            jax.ShapeDtypeStruct((24, B), jnp.float32),
            jax.ShapeDtypeStruct((28, NSEG), jnp.float32),
            jax.ShapeDtypeStruct((LL * 16, B), jnp.float32),
        ),
        mesh=mesh,
        compiler_params=pltpu.CompilerParams(needs_layout_passes=False,
                                             use_tc_tiling_on_sc=False),
        scratch_types=[
            pltpu.VMEM((11, 16), jnp.float32),
            pltpu.VMEM((EXT_PER_W,), jnp.int32),
            pltpu.VMEM((EXT_PER_W,), jnp.int32),
            pltpu.VMEM((EXT_PER_W,), jnp.int32),
            pltpu.VMEM((EXT_PER_W, 16), jnp.float32),
            pltpu.VMEM((EXT_PER_W, 16), jnp.float32),
            pltpu.VMEM((24, EXT_PER_W), jnp.float32),
            pltpu.VMEM((2, CSEG), jnp.int32),
            pltpu.VMEM((2, 4, CSEG), jnp.int32),
            pltpu.VMEM((2, CSEG, 16), jnp.float32),
            pltpu.VMEM((2, 28, CSEG), jnp.float32),
            pltpu.VMEM((EXT_PER_W,), jnp.int32),
            pltpu.VMEM((EXT_PER_W, 16), jnp.float32),
            pltpu.VMEM((16, EXT_PER_W), jnp.float32),
            pltpu.SemaphoreType.DMA,
            pltpu.SemaphoreType.DMA((2,)),
            pltpu.SemaphoreType.DMA((2,)),
            pltpu.SemaphoreType.DMA((2,)),
            pltpu.SemaphoreType.DMA((2,)),
        ],
    )
    return f(weekID, timeID, driverID, seg_idx, if0, if1, if2, if3,
             ff0, ff1, ff2, ff3, cross_idx, delay,
             Wsmall, Wtime, Wdrv, Wseg, Wcross)


def kernel(weekID, timeID, driverID, segID, segment_functional_level,
           roadState, laneNum, roadLevel, wid, speedLimit, time, len,
           crossID, delayTime, W_weekID, W_timeID, W_driverID, W_segID,
           W_segment_functional_level, W_roadState, W_laneNum, W_roadLevel,
           W_crossID):
    def qv(x):
        return x.T.reshape(-1)

    def padcols(w):
        n, d = w.shape
        wp = jnp.pad(w, ((0, 16 - n), (0, 0)))
        return [wp[:, c] for c in range(d)]

    Wsmall = jnp.stack(
        padcols(W_segment_functional_level) + padcols(W_roadState)
        + padcols(W_laneNum) + padcols(W_roadLevel)
        + padcols(W_weekID))

    def rowmajor(w):
        return lax.optimization_barrier(w.reshape(-1)).reshape(w.shape)

    ext_t, seg_t, link_t = _run(
        weekID.astype(jnp.int32), timeID.astype(jnp.int32),
        driverID.astype(jnp.int32), qv(segID.astype(jnp.int32)),
        qv(segment_functional_level.astype(jnp.int32)),
        qv(roadState.astype(jnp.int32)), qv(laneNum.astype(jnp.int32)),
        qv(roadLevel.astype(jnp.int32)),
        qv(wid), qv(speedLimit), qv(time), qv(len),
        qv(crossID.astype(jnp.int32)), qv(delayTime),
        Wsmall, jnp.pad(W_timeID, ((0, 0), (0, 11))), rowmajor(W_driverID),
        rowmajor(W_segID), jnp.pad(W_crossID, ((0, 0), (0, 1))))

    ext = ext_t.T
    seg = seg_t.reshape(28, LS, B).transpose(2, 1, 0)
    link = link_t.reshape(LL, 16, B).transpose(2, 0, 1)
    return ext, seg, link
